# baseline (speedup 1.0000x reference)
import functools

import jax
import jax.numpy as jnp
from jax import lax
from jax.experimental import pallas as pl
from jax.experimental.pallas import tpu as pltpu
from jax.experimental.pallas import tpu_sc as plsc

B = 512
D = 256
N = 4 * B
NUM_NEG = 128
ALPHA = 0.5
TILE = 256
GRID = N // TILE
MASK_FILL = -1.005

NW = 32
RPW = N // NW
NGRP = RPW // 16
NB = 256
BASE1 = -1.01
SCALE1 = NB / 2.02
SCALE2 = NB * SCALE1
KF = float(NUM_NEG)
STEP = 8


def _dense_kernel(emb_ref, w1_ref, b1_ref, w2_ref, b2_ref, dw_ref, pos_ref,
                  sim_ref, m_ref, it_ref, ps_ref, reg_ref, *, base):
    i = pl.program_id(0)

    emb_full = emb_ref[...]
    nrm = jnp.sqrt(jnp.sum(emb_full * emb_full, axis=1, keepdims=True))
    emb_n = emb_full / jnp.maximum(nrm, 1e-12)

    row0 = base + i * TILE
    tile_raw = emb_ref[pl.ds(row0, TILE), :]
    tile_nrm = jnp.sqrt(jnp.sum(tile_raw * tile_raw, axis=1, keepdims=True))
    tile_n = tile_raw / jnp.maximum(tile_nrm, 1e-12)

    sim = lax.dot_general(tile_n, emb_n, (((1,), (1,)), ((), ())),
                          preferred_element_type=jnp.float32)

    h = jnp.maximum(jnp.dot(tile_raw, w1_ref[...],
                            preferred_element_type=jnp.float32)
                    + b1_ref[...], 0.0)
    tlin = jnp.dot(h, w2_ref[...], preferred_element_type=jnp.float32) \
        + b2_ref[...]
    temps = 0.01 + 0.99 * jax.nn.sigmoid(tlin)
    it_ref[...] = 1.0 / temps

    local = row0 % B + lax.broadcasted_iota(jnp.int32, (TILE, 1), 0)
    pr = pos_ref[pl.ds(row0, TILE), :]
    pos_local = pr + (pr >= local).astype(jnp.int32)
    pos_idx = (row0 // B) * B + pos_local

    col = lax.broadcasted_iota(jnp.int32, (TILE, N), 1)
    pos_sim = jnp.sum(jnp.where(col == pos_idx, sim, 0.0), axis=1,
                      keepdims=True)
    ps_ref[...] = pos_sim

    dom = row0 // B
    masked = jnp.where((col // B) == dom, MASK_FILL, sim)
    sim_ref[...] = masked

    row_max = jnp.max(masked, axis=1, keepdims=True)
    m_ref[...] = jnp.maximum(pos_sim, row_max)

    @pl.when(i == 0)
    def _():
        cent = jnp.mean(emb_full.reshape(4, B, D), axis=1)
        reg = jnp.zeros((1, 1), jnp.float32)
        for a in range(4):
            for b in range(a + 1, 4):
                dvec = cent[a] - cent[b]
                reg = reg + dw_ref[a, b] * jnp.sqrt(jnp.sum(dvec * dvec))
        reg_ref[...] = reg / 6.0


TSTRIDE = 17


def _mine_body(sim_hbm, out_hbm, buf_v, bt_v, h1_v, h2_v, out_v, *,
               rpw, ngrp):
    wid = lax.axis_index("s") * 2 + lax.axis_index("c")
    row0 = wid * rpw

    lanes = lax.broadcasted_iota(jnp.int32, (16,), 0)
    ones = jnp.ones((16,), jnp.float32)
    zf = jnp.zeros((16,), jnp.float32)
    zi = jnp.zeros((16,), jnp.int32)

    @plsc.parallel_loop(0, NB, 1, unroll=8)
    def _zero0(b):
        h1_v[pl.ds(b * 16, 16)] = zf

    def per_group(g, _):
        r0 = row0 + g * 16
        pltpu.sync_copy(sim_hbm.at[pl.ds(r0 * N, 16 * N)], buf_v)

        def tpose(j, jv17):
            for l in range(16):
                v = buf_v[pl.ds(l * N + j, 16)]
                plsc.store_scatter(bt_v, [jv17 + l], v)
            return jv17 + 16 * TSTRIDE
        plsc.parallel_loop(0, N, 16, unroll=2,
                           carry=lanes * TSTRIDE)(tpose)

        def pass1(j, js):
            for t in range(STEP):
                v = bt_v[pl.ds(js + t * TSTRIDE, 16)]
                b1 = jnp.minimum(((v - BASE1) * SCALE1).astype(jnp.int32),
                                 NB - 1)
                plsc.addupdate_scatter(h1_v, [(b1 << 4) + lanes], ones)
            return js + STEP * TSTRIDE
        plsc.parallel_loop(0, N, STEP, unroll=4,
                           carry=jnp.int32(0))(pass1)

        def scan1(t, carry):
            cum, bsel, ca = carry
            bb = NB - 1 - t
            hh = h1_v[pl.ds(bb * 16, 16)]
            h2_v[pl.ds(t * 16, 16)] = zf
            new = cum + hh
            hit = (new >= KF) & (cum < KF)
            return new, jnp.where(hit, bb, bsel), jnp.where(hit, cum, ca)
        _, b1sel, ca1 = plsc.parallel_loop(0, NB, 1, unroll=4,
                                           carry=(zf, zi, zf))(scan1)
        lo1 = BASE1 + b1sel.astype(jnp.float32) * (1.0 / SCALE1)

        def pass2(j, js):
            for t in range(STEP):
                v = bt_v[pl.ds(js + t * TSTRIDE, 16)]
                b1 = ((v - BASE1) * SCALE1).astype(jnp.int32)
                b2 = jnp.minimum(((v - lo1) * SCALE2).astype(jnp.int32),
                                 NB - 1)
                plsc.addupdate_scatter(h2_v, [(b2 << 4) + lanes], ones,
                                       mask=b1 == b1sel)
            return js + STEP * TSTRIDE
        plsc.parallel_loop(0, N, STEP, unroll=4,
                           carry=jnp.int32(0))(pass2)

        th = KF - ca1

        def scan2(t, carry):
            cum, bsel = carry
            bb = NB - 1 - t
            hh = h2_v[pl.ds(bb * 16, 16)]
            h1_v[pl.ds(t * 16, 16)] = zf
            new = cum + hh
            hit = (new >= th) & (cum < th)
            return new, jnp.where(hit, bb, bsel)
        _, b2sel = plsc.parallel_loop(0, NB, 1, unroll=4,
                                      carry=(zf, zi))(scan2)

        out_v[pl.ds(g * 16, 16)] = lo1 \
            + (b2sel.astype(jnp.float32) + 1.0) * (1.0 / SCALE2)
        return 0

    lax.fori_loop(0, ngrp, per_group, 0)
    pltpu.sync_copy(out_v, out_hbm.at[pl.ds(row0, rpw)])


def _make_mine(rows):
    rpw = rows // NW
    return functools.partial(
        pl.kernel,
        mesh=plsc.VectorSubcoreMesh(core_axis_name="c", subcore_axis_name="s"),
        out_type=jax.ShapeDtypeStruct((rows,), jnp.float32),
        compiler_params=pltpu.CompilerParams(needs_layout_passes=False),
        scratch_types=[
            pltpu.VMEM((16 * N,), jnp.float32),
            pltpu.VMEM((N * TSTRIDE,), jnp.float32),
            pltpu.VMEM((NB * 16,), jnp.float32),
            pltpu.VMEM((NB * 16,), jnp.float32),
            pltpu.VMEM((rpw,), jnp.float32),
        ],
    )(functools.partial(_mine_body, rpw=rpw, ngrp=rpw // 16))


NCHUNK = 2
CH = N // NCHUNK
_mine_chunk = _make_mine(CH)


def _final_kernel(sim_ref, t_ref, ps_ref, m_ref, it_ref, hw_ref, loss_ref):
    i = pl.program_id(0)
    sim = sim_ref[...]
    t = t_ref[...]
    m = m_ref[...]
    ps = ps_ref[...]
    it = it_ref[...]
    mask = sim > t
    cnt = jnp.sum(mask.astype(jnp.float32), axis=1, keepdims=True)
    ex = jnp.where(mask, jnp.exp((sim - m) * it), 0.0)
    sum_top = jnp.sum(ex, axis=1, keepdims=True) \
        + (KF - cnt) * jnp.exp((t - m) * it)
    total = jnp.exp((ps - m) * it) + sum_top
    losses = ((m - ps) * it + jnp.log(total)) * hw_ref[...]
    part = jnp.sum(losses).reshape(1, 1)

    @pl.when(i == 0)
    def _():
        loss_ref[...] = jnp.zeros((1, 1), jnp.float32)

    loss_ref[...] += part


@jax.jit
def _run(all_emb, w1, b1, w2, b2, dw, hw, pos_rand):
    whole = lambda x: pl.BlockSpec(x.shape, lambda i: (0,) * x.ndim)
    args = (all_emb, w1, b1.reshape(1, 64), w2, b2.reshape(1, 1), dw,
            pos_rand.reshape(N, 1))
    col1 = jax.ShapeDtypeStruct((CH, 1), jnp.float32)
    rowspec = pl.BlockSpec((TILE, 1), lambda i: (i, 0))
    hw2 = hw.reshape(N, 1)

    dense_out = []
    for c in range(NCHUNK):
        dense_out.append(pl.pallas_call(
            functools.partial(_dense_kernel, base=c * CH),
            grid=(CH // TILE,),
            in_specs=[whole(a) for a in args],
            out_specs=[
                pl.BlockSpec((TILE, N), lambda i: (i, 0)),
                rowspec, rowspec, rowspec,
                pl.BlockSpec((1, 1), lambda i: (0, 0)),
            ],
            out_shape=[
                jax.ShapeDtypeStruct((CH, N), jnp.float32),
                col1, col1, col1,
                jax.ShapeDtypeStruct((1, 1), jnp.float32),
            ],
        )(*args))

    t_edges = [_mine_chunk(dense_out[c][0].reshape(-1))
               for c in range(NCHUNK)]

    loss = None
    for c in range(NCHUNK):
        sim, m, it, ps, reg = dense_out[c]
        fargs = (sim, t_edges[c].reshape(CH, 1), ps, m, it,
                 lax.dynamic_slice(hw2, (c * CH, 0), (CH, 1)))
        part = pl.pallas_call(
            _final_kernel,
            grid=(CH // TILE,),
            in_specs=[pl.BlockSpec((TILE, N), lambda i: (i, 0)),
                      rowspec, rowspec, rowspec, rowspec, rowspec],
            out_specs=pl.BlockSpec((1, 1), lambda i: (0, 0)),
            out_shape=jax.ShapeDtypeStruct((1, 1), jnp.float32),
        )(*fargs)
        loss = part if loss is None else loss + part
    return loss[0, 0] / N + ALPHA * dense_out[0][4][0, 0]


def kernel(emb_vision, emb_nlp, emb_security, emb_medical, hard_sample_weights,
           W1, b1, W2, b2, domain_weights, domain_ids, pos_rand):
    all_emb = jnp.concatenate([emb_vision, emb_nlp, emb_security, emb_medical],
                              axis=0)
    return _run(all_emb, W1, b1, W2, b2, domain_weights, hard_sample_weights,
                pos_rand)


# speedup vs baseline: 1.2280x; 1.2280x over previous
import functools

import jax
import jax.numpy as jnp
from jax import lax
from jax.experimental import pallas as pl
from jax.experimental.pallas import tpu as pltpu
from jax.experimental.pallas import tpu_sc as plsc

B = 512
D = 256
N = 4 * B
NUM_NEG = 128
ALPHA = 0.5
TILE = 256
GRID = N // TILE
MASK_FILL = -1.005

NW = 32
RPW = N // NW
NGRP = RPW // 16
NB = 256
BASE1 = -1.01
SCALE1 = NB / 2.02
SCALE2 = NB * SCALE1
KF = float(NUM_NEG)
STEP = 8


def _dense_kernel(emb_ref, w1_ref, b1_ref, w2_ref, b2_ref, dw_ref, pos_ref,
                  sim_ref, m_ref, it_ref, ps_ref, reg_ref):
    i = pl.program_id(0)

    emb_full = emb_ref[...]
    nrm = jnp.sqrt(jnp.sum(emb_full * emb_full, axis=1, keepdims=True))
    emb_n = emb_full / jnp.maximum(nrm, 1e-12)

    row0 = i * TILE
    tile_raw = emb_ref[pl.ds(row0, TILE), :]
    tile_nrm = jnp.sqrt(jnp.sum(tile_raw * tile_raw, axis=1, keepdims=True))
    tile_n = tile_raw / jnp.maximum(tile_nrm, 1e-12)

    sim = lax.dot_general(tile_n, emb_n, (((1,), (1,)), ((), ())),
                          preferred_element_type=jnp.float32)

    h = jnp.maximum(jnp.dot(tile_raw, w1_ref[...],
                            preferred_element_type=jnp.float32)
                    + b1_ref[...], 0.0)
    tlin = jnp.dot(h, w2_ref[...], preferred_element_type=jnp.float32) \
        + b2_ref[...]
    temps = 0.01 + 0.99 * jax.nn.sigmoid(tlin)
    it_ref[...] = 1.0 / temps

    local = row0 % B + lax.broadcasted_iota(jnp.int32, (TILE, 1), 0)
    pr = pos_ref[pl.ds(row0, TILE), :]
    pos_local = pr + (pr >= local).astype(jnp.int32)
    pos_idx = (row0 // B) * B + pos_local

    col = lax.broadcasted_iota(jnp.int32, (TILE, N), 1)
    pos_sim = jnp.sum(jnp.where(col == pos_idx, sim, 0.0), axis=1,
                      keepdims=True)
    ps_ref[...] = pos_sim

    dom = row0 // B
    masked = jnp.where((col // B) == dom, MASK_FILL, sim)
    sim_ref[...] = masked

    row_max = jnp.max(masked, axis=1, keepdims=True)
    m_ref[...] = jnp.maximum(pos_sim, row_max)

    @pl.when(i == 0)
    def _():
        cent = jnp.mean(emb_full.reshape(4, B, D), axis=1)
        reg = jnp.zeros((1, 1), jnp.float32)
        for a in range(4):
            for b in range(a + 1, 4):
                dvec = cent[a] - cent[b]
                reg = reg + dw_ref[a, b] * jnp.sqrt(jnp.sum(dvec * dvec))
        reg_ref[...] = reg / 6.0


TSTRIDE = 17


def _mine_body(sim_hbm, out_hbm, buf_v, bt_v, h1_v, h2_v, out_v, sem):
    wid = lax.axis_index("s") * 2 + lax.axis_index("c")
    row0 = wid * RPW

    lanes = lax.broadcasted_iota(jnp.int32, (16,), 0)
    ones = jnp.ones((16,), jnp.float32)
    zf = jnp.zeros((16,), jnp.float32)
    zi = jnp.zeros((16,), jnp.int32)

    @plsc.parallel_loop(0, NB, 1, unroll=8)
    def _zero0(b):
        h1_v[pl.ds(b * 16, 16)] = zf

    def per_group(g, _):
        r0 = row0 + g * 16
        copies = [pltpu.async_copy(sim_hbm.at[r0 + l, :],
                                   buf_v.at[pl.ds(l * N, N)], sem)
                  for l in range(16)]
        for cp in copies:
            cp.wait()

        def tpose(j, jv17):
            for l in range(16):
                v = buf_v[pl.ds(l * N + j, 16)]
                plsc.store_scatter(bt_v, [jv17 + l], v)
            return jv17 + 16 * TSTRIDE
        plsc.parallel_loop(0, N, 16, unroll=2,
                           carry=lanes * TSTRIDE)(tpose)

        def pass1(j, js):
            for t in range(STEP):
                v = bt_v[pl.ds(js + t * TSTRIDE, 16)]
                b1 = jnp.minimum(((v - BASE1) * SCALE1).astype(jnp.int32),
                                 NB - 1)
                plsc.addupdate_scatter(h1_v, [(b1 << 4) + lanes], ones)
            return js + STEP * TSTRIDE
        plsc.parallel_loop(0, N, STEP, unroll=4,
                           carry=jnp.int32(0))(pass1)

        def scan1(t, carry):
            cum, bsel, ca = carry
            bb = NB - 1 - t
            hh = h1_v[pl.ds(bb * 16, 16)]
            h2_v[pl.ds(t * 16, 16)] = zf
            new = cum + hh
            hit = (new >= KF) & (cum < KF)
            return new, jnp.where(hit, bb, bsel), jnp.where(hit, cum, ca)
        _, b1sel, ca1 = plsc.parallel_loop(0, NB, 1, unroll=4,
                                           carry=(zf, zi, zf))(scan1)
        lo1 = BASE1 + b1sel.astype(jnp.float32) * (1.0 / SCALE1)

        def pass2(j, js):
            for t in range(STEP):
                v = bt_v[pl.ds(js + t * TSTRIDE, 16)]
                b1 = ((v - BASE1) * SCALE1).astype(jnp.int32)
                b2 = jnp.minimum(((v - lo1) * SCALE2).astype(jnp.int32),
                                 NB - 1)
                plsc.addupdate_scatter(h2_v, [(b2 << 4) + lanes], ones,
                                       mask=b1 == b1sel)
            return js + STEP * TSTRIDE
        plsc.parallel_loop(0, N, STEP, unroll=4,
                           carry=jnp.int32(0))(pass2)

        th = KF - ca1

        def scan2(t, carry):
            cum, bsel = carry
            bb = NB - 1 - t
            hh = h2_v[pl.ds(bb * 16, 16)]
            h1_v[pl.ds(t * 16, 16)] = zf
            new = cum + hh
            hit = (new >= th) & (cum < th)
            return new, jnp.where(hit, bb, bsel)
        _, b2sel = plsc.parallel_loop(0, NB, 1, unroll=4,
                                      carry=(zf, zi))(scan2)

        out_v[pl.ds(g * 16, 16)] = lo1 \
            + (b2sel.astype(jnp.float32) + 1.0) * (1.0 / SCALE2)
        return 0

    lax.fori_loop(0, NGRP, per_group, 0)
    pltpu.sync_copy(out_v, out_hbm.at[pl.ds(row0, RPW)])


_mine = functools.partial(
    pl.kernel,
    mesh=plsc.VectorSubcoreMesh(core_axis_name="c", subcore_axis_name="s"),
    out_type=jax.ShapeDtypeStruct((N,), jnp.float32),
    compiler_params=pltpu.CompilerParams(needs_layout_passes=False),
    scratch_types=[
        pltpu.VMEM((16 * N,), jnp.float32),
        pltpu.VMEM((N * TSTRIDE,), jnp.float32),
        pltpu.VMEM((NB * 16,), jnp.float32),
        pltpu.VMEM((NB * 16,), jnp.float32),
        pltpu.VMEM((RPW,), jnp.float32),
        pltpu.SemaphoreType.DMA,
    ],
)(_mine_body)


def _final_kernel(sim_ref, t_ref, ps_ref, m_ref, it_ref, hw_ref, loss_ref):
    i = pl.program_id(0)
    sim = sim_ref[...]
    t = t_ref[...]
    m = m_ref[...]
    ps = ps_ref[...]
    it = it_ref[...]
    mask = sim > t
    cnt = jnp.sum(mask.astype(jnp.float32), axis=1, keepdims=True)
    ex = jnp.where(mask, jnp.exp((sim - m) * it), 0.0)
    sum_top = jnp.sum(ex, axis=1, keepdims=True) \
        + (KF - cnt) * jnp.exp((t - m) * it)
    total = jnp.exp((ps - m) * it) + sum_top
    losses = ((m - ps) * it + jnp.log(total)) * hw_ref[...]
    part = jnp.sum(losses).reshape(1, 1)

    @pl.when(i == 0)
    def _():
        loss_ref[...] = jnp.zeros((1, 1), jnp.float32)

    loss_ref[...] += part


@jax.jit
def _run(all_emb, w1, b1, w2, b2, dw, hw, pos_rand):
    whole = lambda x: pl.BlockSpec(x.shape, lambda i: (0,) * x.ndim)
    args = (all_emb, w1, b1.reshape(1, 64), w2, b2.reshape(1, 1), dw,
            pos_rand.reshape(N, 1))
    col1 = jax.ShapeDtypeStruct((N, 1), jnp.float32)
    rowspec = pl.BlockSpec((TILE, 1), lambda i: (i, 0))
    sim, m, it, ps, reg = pl.pallas_call(
        _dense_kernel,
        grid=(GRID,),
        in_specs=[whole(a) for a in args],
        out_specs=[
            pl.BlockSpec((TILE, N), lambda i: (i, 0)),
            rowspec, rowspec, rowspec,
            pl.BlockSpec((1, 1), lambda i: (0, 0)),
        ],
        out_shape=[
            jax.ShapeDtypeStruct((N, N), jnp.float32),
            col1, col1, col1,
            jax.ShapeDtypeStruct((1, 1), jnp.float32),
        ],
    )(*args)

    t_edge = _mine(sim)

    fargs = (sim, t_edge.reshape(N, 1), ps, m, it, hw.reshape(N, 1))
    loss_sum = pl.pallas_call(
        _final_kernel,
        grid=(GRID,),
        in_specs=[pl.BlockSpec((TILE, N), lambda i: (i, 0)),
                  rowspec, rowspec, rowspec, rowspec, rowspec],
        out_specs=pl.BlockSpec((1, 1), lambda i: (0, 0)),
        out_shape=jax.ShapeDtypeStruct((1, 1), jnp.float32),
    )(*fargs)
    return loss_sum[0, 0] / N + ALPHA * reg[0, 0]


def kernel(emb_vision, emb_nlp, emb_security, emb_medical, hard_sample_weights,
           W1, b1, W2, b2, domain_weights, domain_ids, pos_rand):
    all_emb = jnp.concatenate([emb_vision, emb_nlp, emb_security, emb_medical],
                              axis=0)
    return _run(all_emb, W1, b1, W2, b2, domain_weights, hard_sample_weights,
                pos_rand)


# speedup vs baseline: 1.3628x; 1.1097x over previous
import functools

import jax
import jax.numpy as jnp
from jax import lax
from jax.experimental import pallas as pl
from jax.experimental.pallas import tpu as pltpu
from jax.experimental.pallas import tpu_sc as plsc

B = 512
D = 256
N = 4 * B
NUM_NEG = 128
ALPHA = 0.5
TILE = 256
GRID = N // TILE
MASK_FILL = -1.005

NW = 32
RPW = N // NW
NGRP = RPW // 16
NB = 256
BASE1 = -1.01
SCALE1 = NB / 2.02
SCALE2 = NB * SCALE1
KF = float(NUM_NEG)
STEP = 8


def _dense_kernel(emb_ref, w1_ref, b1_ref, w2_ref, b2_ref, dw_ref, pos_ref,
                  sim_ref, m_ref, it_ref, ps_ref, reg_ref):
    i = pl.program_id(0)

    emb_full = emb_ref[...]
    nrm = jnp.sqrt(jnp.sum(emb_full * emb_full, axis=1, keepdims=True))
    emb_n = emb_full / jnp.maximum(nrm, 1e-12)

    row0 = i * TILE
    tile_raw = emb_ref[pl.ds(row0, TILE), :]
    tile_nrm = jnp.sqrt(jnp.sum(tile_raw * tile_raw, axis=1, keepdims=True))
    tile_n = tile_raw / jnp.maximum(tile_nrm, 1e-12)

    sim = lax.dot_general(tile_n, emb_n, (((1,), (1,)), ((), ())),
                          preferred_element_type=jnp.float32)

    h = jnp.maximum(jnp.dot(tile_raw, w1_ref[...],
                            preferred_element_type=jnp.float32)
                    + b1_ref[...], 0.0)
    tlin = jnp.dot(h, w2_ref[...], preferred_element_type=jnp.float32) \
        + b2_ref[...]
    temps = 0.01 + 0.99 * jax.nn.sigmoid(tlin)
    it_ref[...] = 1.0 / temps

    local = row0 % B + lax.broadcasted_iota(jnp.int32, (TILE, 1), 0)
    pr = pos_ref[pl.ds(row0, TILE), :]
    pos_local = pr + (pr >= local).astype(jnp.int32)
    pos_idx = (row0 // B) * B + pos_local

    col = lax.broadcasted_iota(jnp.int32, (TILE, N), 1)
    pos_sim = jnp.sum(jnp.where(col == pos_idx, sim, 0.0), axis=1,
                      keepdims=True)
    ps_ref[...] = pos_sim

    dom = row0 // B
    masked = jnp.where((col // B) == dom, MASK_FILL, sim)
    sim_ref[...] = masked

    row_max = jnp.max(masked, axis=1, keepdims=True)
    m_ref[...] = jnp.maximum(pos_sim, row_max)

    @pl.when(i == 0)
    def _():
        cent = jnp.mean(emb_full.reshape(4, B, D), axis=1)
        reg = jnp.zeros((1, 1), jnp.float32)
        for a in range(4):
            for b in range(a + 1, 4):
                dvec = cent[a] - cent[b]
                reg = reg + dw_ref[a, b] * jnp.sqrt(jnp.sum(dvec * dvec))
        reg_ref[...] = reg / 6.0


TSTRIDE = 17


NC_V = 3 * B


def _mine_body(sim_hbm, out_hbm, buf_v, bt_v, h1_v, h2_v, out_v, sem):
    wid = lax.axis_index("s") * 2 + lax.axis_index("c")
    row0 = wid * RPW
    dom = wid // (NW // 4)

    lanes = lax.broadcasted_iota(jnp.int32, (16,), 0)
    ones = jnp.ones((16,), jnp.float32)
    zf = jnp.zeros((16,), jnp.float32)
    zi = jnp.zeros((16,), jnp.int32)

    chunks = [k + jnp.where(k >= dom, 1, 0) for k in range(3)]

    @plsc.parallel_loop(0, NB, 1, unroll=8)
    def _zero0(b):
        h1_v[pl.ds(b * 16, 16)] = zf

    def per_group(g, _):
        r0 = row0 + g * 16
        copies = [pltpu.async_copy(sim_hbm.at[r0 + l, pl.ds(ck * B, B)],
                                   buf_v.at[pl.ds(l * NC_V + k * B, B)], sem)
                  for l in range(16) for k, ck in enumerate(chunks)]
        for cp in copies:
            cp.wait()

        def tpose(j, jv17):
            for l in range(16):
                v = buf_v[pl.ds(l * NC_V + j, 16)]
                plsc.store_scatter(bt_v, [jv17 + l], v)
            return jv17 + 16 * TSTRIDE
        plsc.parallel_loop(0, NC_V, 16, unroll=2,
                           carry=lanes * TSTRIDE)(tpose)

        def pass1(j, js):
            for t in range(STEP):
                v = bt_v[pl.ds(js + t * TSTRIDE, 16)]
                b1 = jnp.minimum(((v - BASE1) * SCALE1).astype(jnp.int32),
                                 NB - 1)
                plsc.addupdate_scatter(h1_v, [(b1 << 4) + lanes], ones)
            return js + STEP * TSTRIDE
        plsc.parallel_loop(0, NC_V, STEP, unroll=4,
                           carry=jnp.int32(0))(pass1)

        def scan1(t, carry):
            cum, bsel, ca = carry
            bb = NB - 1 - t
            hh = h1_v[pl.ds(bb * 16, 16)]
            h2_v[pl.ds(t * 16, 16)] = zf
            new = cum + hh
            hit = (new >= KF) & (cum < KF)
            return new, jnp.where(hit, bb, bsel), jnp.where(hit, cum, ca)
        _, b1sel, ca1 = plsc.parallel_loop(0, NB, 1, unroll=4,
                                           carry=(zf, zi, zf))(scan1)
        lo1 = BASE1 + b1sel.astype(jnp.float32) * (1.0 / SCALE1)

        def pass2(j, js):
            for t in range(STEP):
                v = bt_v[pl.ds(js + t * TSTRIDE, 16)]
                b1 = ((v - BASE1) * SCALE1).astype(jnp.int32)
                b2 = jnp.minimum(((v - lo1) * SCALE2).astype(jnp.int32),
                                 NB - 1)
                plsc.addupdate_scatter(h2_v, [(b2 << 4) + lanes], ones,
                                       mask=b1 == b1sel)
            return js + STEP * TSTRIDE
        plsc.parallel_loop(0, NC_V, STEP, unroll=4,
                           carry=jnp.int32(0))(pass2)

        th = KF - ca1

        def scan2(t, carry):
            cum, bsel = carry
            bb = NB - 1 - t
            hh = h2_v[pl.ds(bb * 16, 16)]
            h1_v[pl.ds(t * 16, 16)] = zf
            new = cum + hh
            hit = (new >= th) & (cum < th)
            return new, jnp.where(hit, bb, bsel)
        _, b2sel = plsc.parallel_loop(0, NB, 1, unroll=4,
                                      carry=(zf, zi))(scan2)

        out_v[pl.ds(g * 16, 16)] = lo1 \
            + (b2sel.astype(jnp.float32) + 1.0) * (1.0 / SCALE2)
        return 0

    lax.fori_loop(0, NGRP, per_group, 0)
    pltpu.sync_copy(out_v, out_hbm.at[pl.ds(row0, RPW)])


_mine = functools.partial(
    pl.kernel,
    mesh=plsc.VectorSubcoreMesh(core_axis_name="c", subcore_axis_name="s"),
    out_type=jax.ShapeDtypeStruct((N,), jnp.float32),
    compiler_params=pltpu.CompilerParams(needs_layout_passes=False),
    scratch_types=[
        pltpu.VMEM((16 * NC_V,), jnp.float32),
        pltpu.VMEM((NC_V * TSTRIDE,), jnp.float32),
        pltpu.VMEM((NB * 16,), jnp.float32),
        pltpu.VMEM((NB * 16,), jnp.float32),
        pltpu.VMEM((RPW,), jnp.float32),
        pltpu.SemaphoreType.DMA,
    ],
)(_mine_body)


def _final_kernel(sim_ref, t_ref, ps_ref, m_ref, it_ref, hw_ref, loss_ref):
    i = pl.program_id(0)
    sim = sim_ref[...]
    t = t_ref[...]
    m = m_ref[...]
    ps = ps_ref[...]
    it = it_ref[...]
    mask = sim > t
    cnt = jnp.sum(mask.astype(jnp.float32), axis=1, keepdims=True)
    ex = jnp.where(mask, jnp.exp((sim - m) * it), 0.0)
    sum_top = jnp.sum(ex, axis=1, keepdims=True) \
        + (KF - cnt) * jnp.exp((t - m) * it)
    total = jnp.exp((ps - m) * it) + sum_top
    losses = ((m - ps) * it + jnp.log(total)) * hw_ref[...]
    part = jnp.sum(losses).reshape(1, 1)

    @pl.when(i == 0)
    def _():
        loss_ref[...] = jnp.zeros((1, 1), jnp.float32)

    loss_ref[...] += part


@jax.jit
def _run(all_emb, w1, b1, w2, b2, dw, hw, pos_rand):
    whole = lambda x: pl.BlockSpec(x.shape, lambda i: (0,) * x.ndim)
    args = (all_emb, w1, b1.reshape(1, 64), w2, b2.reshape(1, 1), dw,
            pos_rand.reshape(N, 1))
    col1 = jax.ShapeDtypeStruct((N, 1), jnp.float32)
    rowspec = pl.BlockSpec((TILE, 1), lambda i: (i, 0))
    sim, m, it, ps, reg = pl.pallas_call(
        _dense_kernel,
        grid=(GRID,),
        in_specs=[whole(a) for a in args],
        out_specs=[
            pl.BlockSpec((TILE, N), lambda i: (i, 0)),
            rowspec, rowspec, rowspec,
            pl.BlockSpec((1, 1), lambda i: (0, 0)),
        ],
        out_shape=[
            jax.ShapeDtypeStruct((N, N), jnp.float32),
            col1, col1, col1,
            jax.ShapeDtypeStruct((1, 1), jnp.float32),
        ],
    )(*args)

    t_edge = _mine(sim)

    fargs = (sim, t_edge.reshape(N, 1), ps, m, it, hw.reshape(N, 1))
    loss_sum = pl.pallas_call(
        _final_kernel,
        grid=(GRID,),
        in_specs=[pl.BlockSpec((TILE, N), lambda i: (i, 0)),
                  rowspec, rowspec, rowspec, rowspec, rowspec],
        out_specs=pl.BlockSpec((1, 1), lambda i: (0, 0)),
        out_shape=jax.ShapeDtypeStruct((1, 1), jnp.float32),
    )(*fargs)
    return loss_sum[0, 0] / N + ALPHA * reg[0, 0]


def kernel(emb_vision, emb_nlp, emb_security, emb_medical, hard_sample_weights,
           W1, b1, W2, b2, domain_weights, domain_ids, pos_rand):
    all_emb = jnp.concatenate([emb_vision, emb_nlp, emb_security, emb_medical],
                              axis=0)
    return _run(all_emb, W1, b1, W2, b2, domain_weights, hard_sample_weights,
                pos_rand)


# speedup vs baseline: 1.3710x; 1.0060x over previous
import functools

import jax
import jax.numpy as jnp
from jax import lax
from jax.experimental import pallas as pl
from jax.experimental.pallas import tpu as pltpu
from jax.experimental.pallas import tpu_sc as plsc

B = 512
D = 256
N = 4 * B
NUM_NEG = 128
ALPHA = 0.5
TILE = 256
GRID = N // TILE
MASK_FILL = -1.005

NW = 32
RPW = N // NW
NGRP = RPW // 16
NB = 256
BASE1 = -1.01
SCALE1 = NB / 2.02
SCALE2 = NB * SCALE1
KF = float(NUM_NEG)
STEP = 8


def _dense_kernel(emb_ref, w1_ref, b1_ref, w2_ref, b2_ref, dw_ref, pos_ref,
                  sim_ref, m_ref, it_ref, ps_ref, reg_ref):
    i = pl.program_id(0)

    emb_full = emb_ref[...]
    nrm = jnp.sqrt(jnp.sum(emb_full * emb_full, axis=1, keepdims=True))
    emb_n = emb_full / jnp.maximum(nrm, 1e-12)

    row0 = i * TILE
    tile_raw = emb_ref[pl.ds(row0, TILE), :]
    tile_nrm = jnp.sqrt(jnp.sum(tile_raw * tile_raw, axis=1, keepdims=True))
    tile_n = tile_raw / jnp.maximum(tile_nrm, 1e-12)

    sim = lax.dot_general(tile_n, emb_n, (((1,), (1,)), ((), ())),
                          preferred_element_type=jnp.float32)

    h = jnp.maximum(jnp.dot(tile_raw, w1_ref[...],
                            preferred_element_type=jnp.float32)
                    + b1_ref[...], 0.0)
    tlin = jnp.dot(h, w2_ref[...], preferred_element_type=jnp.float32) \
        + b2_ref[...]
    temps = 0.01 + 0.99 * jax.nn.sigmoid(tlin)
    it_ref[...] = 1.0 / temps

    local = row0 % B + lax.broadcasted_iota(jnp.int32, (TILE, 1), 0)
    pr = pos_ref[pl.ds(row0, TILE), :]
    pos_local = pr + (pr >= local).astype(jnp.int32)
    pos_idx = (row0 // B) * B + pos_local

    col = lax.broadcasted_iota(jnp.int32, (TILE, N), 1)
    pos_sim = jnp.sum(jnp.where(col == pos_idx, sim, 0.0), axis=1,
                      keepdims=True)
    ps_ref[...] = pos_sim

    dom = row0 // B
    masked = jnp.where((col // B) == dom, MASK_FILL, sim)
    sim_ref[...] = masked

    row_max = jnp.max(masked, axis=1, keepdims=True)
    m_ref[...] = jnp.maximum(pos_sim, row_max)

    @pl.when(i == 0)
    def _():
        cent = jnp.mean(emb_full.reshape(4, B, D), axis=1)
        reg = jnp.zeros((1, 1), jnp.float32)
        for a in range(4):
            for b in range(a + 1, 4):
                dvec = cent[a] - cent[b]
                reg = reg + dw_ref[a, b] * jnp.sqrt(jnp.sum(dvec * dvec))
        reg_ref[...] = reg / 6.0


TSTRIDE = 17


NC_V = 3 * B


def _mine_body(sim_hbm, out_hbm, buf_v, bt_v, h1_v, h2_v, out_v, sem):
    wid = lax.axis_index("s") * 2 + lax.axis_index("c")
    row0 = wid * RPW
    dom = wid // (NW // 4)

    lanes = lax.broadcasted_iota(jnp.int32, (16,), 0)
    ones = jnp.ones((16,), jnp.float32)
    zf = jnp.zeros((16,), jnp.float32)
    zi = jnp.zeros((16,), jnp.int32)

    chunks = [k + jnp.where(k >= dom, 1, 0) for k in range(3)]

    @plsc.parallel_loop(0, NB, 1, unroll=8)
    def _zero0(b):
        h1_v[pl.ds(b * 16, 16)] = zf

    def per_group(g, _):
        r0 = row0 + g * 16
        copies = [pltpu.async_copy(sim_hbm.at[r0 + l, pl.ds(ck * B, B)],
                                   buf_v.at[pl.ds(l * NC_V + k * B, B)], sem)
                  for l in range(16) for k, ck in enumerate(chunks)]
        for cp in copies:
            cp.wait()

        def tpose(j, jv17):
            for l in range(16):
                v = buf_v[pl.ds(l * NC_V + j, 16)]
                plsc.store_scatter(bt_v, [jv17 + l], v)
            return jv17 + 16 * TSTRIDE
        plsc.parallel_loop(0, NC_V, 16, unroll=2,
                           carry=lanes * TSTRIDE)(tpose)

        def pass1(j, js):
            for t in range(STEP):
                v = bt_v[pl.ds(js + t * TSTRIDE, 16)]
                b1 = jnp.minimum(((v - BASE1) * SCALE1).astype(jnp.int32),
                                 NB - 1)
                plsc.addupdate_scatter(h1_v, [(b1 << 4) + lanes], ones)
            return js + STEP * TSTRIDE
        plsc.parallel_loop(0, NC_V, STEP, unroll=8,
                           carry=jnp.int32(0))(pass1)

        def scan1(t, carry):
            cum, bsel, ca = carry
            bb = NB - 1 - t
            hh = h1_v[pl.ds(bb * 16, 16)]
            h2_v[pl.ds(t * 16, 16)] = zf
            new = cum + hh
            hit = (new >= KF) & (cum < KF)
            return new, jnp.where(hit, bb, bsel), jnp.where(hit, cum, ca)
        _, b1sel, ca1 = plsc.parallel_loop(0, NB, 1, unroll=4,
                                           carry=(zf, zi, zf))(scan1)
        lo1 = BASE1 + b1sel.astype(jnp.float32) * (1.0 / SCALE1)

        def pass2(j, js):
            for t in range(STEP):
                v = bt_v[pl.ds(js + t * TSTRIDE, 16)]
                b1 = ((v - BASE1) * SCALE1).astype(jnp.int32)
                b2 = jnp.minimum(((v - lo1) * SCALE2).astype(jnp.int32),
                                 NB - 1)
                plsc.addupdate_scatter(h2_v, [(b2 << 4) + lanes], ones,
                                       mask=b1 == b1sel)
            return js + STEP * TSTRIDE
        plsc.parallel_loop(0, NC_V, STEP, unroll=8,
                           carry=jnp.int32(0))(pass2)

        th = KF - ca1

        def scan2(t, carry):
            cum, bsel = carry
            bb = NB - 1 - t
            hh = h2_v[pl.ds(bb * 16, 16)]
            h1_v[pl.ds(t * 16, 16)] = zf
            new = cum + hh
            hit = (new >= th) & (cum < th)
            return new, jnp.where(hit, bb, bsel)
        _, b2sel = plsc.parallel_loop(0, NB, 1, unroll=4,
                                      carry=(zf, zi))(scan2)

        out_v[pl.ds(g * 16, 16)] = lo1 \
            + (b2sel.astype(jnp.float32) + 1.0) * (1.0 / SCALE2)
        return 0

    lax.fori_loop(0, NGRP, per_group, 0)
    pltpu.sync_copy(out_v, out_hbm.at[pl.ds(row0, RPW)])


_mine = functools.partial(
    pl.kernel,
    mesh=plsc.VectorSubcoreMesh(core_axis_name="c", subcore_axis_name="s"),
    out_type=jax.ShapeDtypeStruct((N,), jnp.float32),
    compiler_params=pltpu.CompilerParams(needs_layout_passes=False),
    scratch_types=[
        pltpu.VMEM((16 * NC_V,), jnp.float32),
        pltpu.VMEM((NC_V * TSTRIDE,), jnp.float32),
        pltpu.VMEM((NB * 16,), jnp.float32),
        pltpu.VMEM((NB * 16,), jnp.float32),
        pltpu.VMEM((RPW,), jnp.float32),
        pltpu.SemaphoreType.DMA,
    ],
)(_mine_body)


def _final_kernel(sim_ref, t_ref, ps_ref, m_ref, it_ref, hw_ref, loss_ref):
    i = pl.program_id(0)
    sim = sim_ref[...]
    t = t_ref[...]
    m = m_ref[...]
    ps = ps_ref[...]
    it = it_ref[...]
    mask = sim > t
    cnt = jnp.sum(mask.astype(jnp.float32), axis=1, keepdims=True)
    ex = jnp.where(mask, jnp.exp((sim - m) * it), 0.0)
    sum_top = jnp.sum(ex, axis=1, keepdims=True) \
        + (KF - cnt) * jnp.exp((t - m) * it)
    total = jnp.exp((ps - m) * it) + sum_top
    losses = ((m - ps) * it + jnp.log(total)) * hw_ref[...]
    part = jnp.sum(losses).reshape(1, 1)

    @pl.when(i == 0)
    def _():
        loss_ref[...] = jnp.zeros((1, 1), jnp.float32)

    loss_ref[...] += part


@jax.jit
def _run(all_emb, w1, b1, w2, b2, dw, hw, pos_rand):
    whole = lambda x: pl.BlockSpec(x.shape, lambda i: (0,) * x.ndim)
    args = (all_emb, w1, b1.reshape(1, 64), w2, b2.reshape(1, 1), dw,
            pos_rand.reshape(N, 1))
    col1 = jax.ShapeDtypeStruct((N, 1), jnp.float32)
    rowspec = pl.BlockSpec((TILE, 1), lambda i: (i, 0))
    sim, m, it, ps, reg = pl.pallas_call(
        _dense_kernel,
        grid=(GRID,),
        in_specs=[whole(a) for a in args],
        out_specs=[
            pl.BlockSpec((TILE, N), lambda i: (i, 0)),
            rowspec, rowspec, rowspec,
            pl.BlockSpec((1, 1), lambda i: (0, 0)),
        ],
        out_shape=[
            jax.ShapeDtypeStruct((N, N), jnp.float32),
            col1, col1, col1,
            jax.ShapeDtypeStruct((1, 1), jnp.float32),
        ],
    )(*args)

    t_edge = _mine(sim)

    fargs = (sim, t_edge.reshape(N, 1), ps, m, it, hw.reshape(N, 1))
    loss_sum = pl.pallas_call(
        _final_kernel,
        grid=(GRID,),
        in_specs=[pl.BlockSpec((TILE, N), lambda i: (i, 0)),
                  rowspec, rowspec, rowspec, rowspec, rowspec],
        out_specs=pl.BlockSpec((1, 1), lambda i: (0, 0)),
        out_shape=jax.ShapeDtypeStruct((1, 1), jnp.float32),
    )(*fargs)
    return loss_sum[0, 0] / N + ALPHA * reg[0, 0]


def kernel(emb_vision, emb_nlp, emb_security, emb_medical, hard_sample_weights,
           W1, b1, W2, b2, domain_weights, domain_ids, pos_rand):
    all_emb = jnp.concatenate([emb_vision, emb_nlp, emb_security, emb_medical],
                              axis=0)
    return _run(all_emb, W1, b1, W2, b2, domain_weights, hard_sample_weights,
                pos_rand)


# speedup vs baseline: 1.3739x; 1.0021x over previous
import functools

import jax
import jax.numpy as jnp
from jax import lax
from jax.experimental import pallas as pl
from jax.experimental.pallas import tpu as pltpu
from jax.experimental.pallas import tpu_sc as plsc

B = 512
D = 256
N = 4 * B
NUM_NEG = 128
ALPHA = 0.5
TILE = 256
GRID = N // TILE
MASK_FILL = -1.005

NW = 32
RPW = N // NW
NGRP = RPW // 16
NB = 256
BASE1 = -1.01
SCALE1 = NB / 2.02
SCALE2 = NB * SCALE1
KF = float(NUM_NEG)
STEP = 8


def _dense_kernel(emb_ref, w1_ref, b1_ref, w2_ref, b2_ref, dw_ref, pos_ref,
                  sim_ref, m_ref, it_ref, ps_ref, reg_ref):
    i = pl.program_id(0)

    emb_full = emb_ref[...]
    nrm = jnp.sqrt(jnp.sum(emb_full * emb_full, axis=1, keepdims=True))
    emb_n = emb_full / jnp.maximum(nrm, 1e-12)

    row0 = i * TILE
    tile_raw = emb_ref[pl.ds(row0, TILE), :]
    tile_nrm = jnp.sqrt(jnp.sum(tile_raw * tile_raw, axis=1, keepdims=True))
    tile_n = tile_raw / jnp.maximum(tile_nrm, 1e-12)

    sim = lax.dot_general(tile_n, emb_n, (((1,), (1,)), ((), ())),
                          preferred_element_type=jnp.float32)

    h = jnp.maximum(jnp.dot(tile_raw, w1_ref[...],
                            preferred_element_type=jnp.float32)
                    + b1_ref[...], 0.0)
    tlin = jnp.dot(h, w2_ref[...], preferred_element_type=jnp.float32) \
        + b2_ref[...]
    temps = 0.01 + 0.99 * jax.nn.sigmoid(tlin)
    it_ref[...] = 1.0 / temps

    local = row0 % B + lax.broadcasted_iota(jnp.int32, (TILE, 1), 0)
    pr = pos_ref[pl.ds(row0, TILE), :]
    pos_local = pr + (pr >= local).astype(jnp.int32)
    pos_idx = (row0 // B) * B + pos_local

    col = lax.broadcasted_iota(jnp.int32, (TILE, N), 1)
    pos_sim = jnp.sum(jnp.where(col == pos_idx, sim, 0.0), axis=1,
                      keepdims=True)
    ps_ref[...] = pos_sim

    dom = row0 // B
    masked = jnp.where((col // B) == dom, MASK_FILL, sim)
    sim_ref[...] = masked

    row_max = jnp.max(masked, axis=1, keepdims=True)
    m_ref[...] = jnp.maximum(pos_sim, row_max)

    @pl.when(i == 0)
    def _():
        cent = jnp.mean(emb_full.reshape(4, B, D), axis=1)
        reg = jnp.zeros((1, 1), jnp.float32)
        for a in range(4):
            for b in range(a + 1, 4):
                dvec = cent[a] - cent[b]
                reg = reg + dw_ref[a, b] * jnp.sqrt(jnp.sum(dvec * dvec))
        reg_ref[...] = reg / 6.0


TSTRIDE = 17


NC_V = 3 * B


def _mine_body(sim_hbm, out_hbm, buf_v, bt_v, h1_v, h2_v, out_v, sem):
    wid = lax.axis_index("s") * 2 + lax.axis_index("c")
    row0 = wid * RPW
    dom = wid // (NW // 4)

    lanes = lax.broadcasted_iota(jnp.int32, (16,), 0)
    ones = jnp.ones((16,), jnp.float32)
    zf = jnp.zeros((16,), jnp.float32)
    zi = jnp.zeros((16,), jnp.int32)

    chunks = [k + jnp.where(k >= dom, 1, 0) for k in range(3)]

    @plsc.parallel_loop(0, NB, 1, unroll=8)
    def _zero0(b):
        h1_v[pl.ds(b * 16, 16)] = zf

    def per_group(g, _):
        r0 = row0 + g * 16
        copies = [pltpu.async_copy(sim_hbm.at[r0 + l, pl.ds(ck * B, B)],
                                   buf_v.at[pl.ds(l * NC_V + k * B, B)], sem)
                  for l in range(16) for k, ck in enumerate(chunks)]
        for cp in copies:
            cp.wait()

        def tpose(j, jv17):
            for l in range(16):
                v = buf_v[pl.ds(l * NC_V + j, 16)]
                plsc.store_scatter(bt_v, [jv17 + l], v)
            return jv17 + 16 * TSTRIDE
        plsc.parallel_loop(0, NC_V, 16, unroll=2,
                           carry=lanes * TSTRIDE)(tpose)

        def pass1(j, js):
            for t in range(STEP):
                v = bt_v[pl.ds(js + t * TSTRIDE, 16)]
                b1 = jnp.minimum(((v - BASE1) * SCALE1).astype(jnp.int32),
                                 NB - 1)
                plsc.addupdate_scatter(h1_v, [(b1 << 4) + lanes], ones)
            return js + STEP * TSTRIDE
        plsc.parallel_loop(0, NC_V, STEP, unroll=8,
                           carry=jnp.int32(0))(pass1)

        def scan1(t, carry):
            cum, bsel, ca = carry
            bb = NB - 1 - t
            hh = h1_v[pl.ds(bb * 16, 16)]
            h2_v[pl.ds(t * 16, 16)] = zf
            new = cum + hh
            hit = (new >= KF) & (cum < KF)
            return new, jnp.where(hit, bb, bsel), jnp.where(hit, cum, ca)
        _, b1sel, ca1 = plsc.parallel_loop(0, NB, 1, unroll=4,
                                           carry=(zf, zi, zf))(scan1)
        lo1 = BASE1 + b1sel.astype(jnp.float32) * (1.0 / SCALE1)

        def pass2(j, js):
            for t in range(STEP):
                v = bt_v[pl.ds(js + t * TSTRIDE, 16)]
                b1 = ((v - BASE1) * SCALE1).astype(jnp.int32)
                b2 = jnp.minimum(((v - lo1) * SCALE2).astype(jnp.int32),
                                 NB - 1)
                plsc.addupdate_scatter(h2_v, [(b2 << 4) + lanes], ones,
                                       mask=b1 == b1sel)
            return js + STEP * TSTRIDE
        plsc.parallel_loop(0, NC_V, STEP, unroll=8,
                           carry=jnp.int32(0))(pass2)

        th = KF - ca1

        def scan2(t, carry):
            cum, bsel = carry
            bb = NB - 1 - t
            hh = h2_v[pl.ds(bb * 16, 16)]
            h1_v[pl.ds(t * 16, 16)] = zf
            new = cum + hh
            hit = (new >= th) & (cum < th)
            return new, jnp.where(hit, bb, bsel)
        _, b2sel = plsc.parallel_loop(0, NB, 1, unroll=4,
                                      carry=(zf, zi))(scan2)

        out_v[pl.ds(g * 16, 16)] = lo1 \
            + (b2sel.astype(jnp.float32) + 1.0) * (1.0 / SCALE2)
        return 0

    lax.fori_loop(0, NGRP, per_group, 0)
    pltpu.sync_copy(out_v, out_hbm.at[pl.ds(row0, RPW)])


_mine = functools.partial(
    pl.kernel,
    mesh=plsc.VectorSubcoreMesh(core_axis_name="c", subcore_axis_name="s",
                                num_cores=2, num_subcores=16),
    out_type=jax.ShapeDtypeStruct((N,), jnp.float32),
    compiler_params=pltpu.CompilerParams(needs_layout_passes=False),
    scratch_types=[
        pltpu.VMEM((16 * NC_V,), jnp.float32),
        pltpu.VMEM((NC_V * TSTRIDE,), jnp.float32),
        pltpu.VMEM((NB * 16,), jnp.float32),
        pltpu.VMEM((NB * 16,), jnp.float32),
        pltpu.VMEM((RPW,), jnp.float32),
        pltpu.SemaphoreType.DMA,
    ],
)(_mine_body)


def _final_kernel(sim_ref, t_ref, ps_ref, m_ref, it_ref, hw_ref, loss_ref):
    i = pl.program_id(0)
    sim = sim_ref[...]
    t = t_ref[...]
    m = m_ref[...]
    ps = ps_ref[...]
    it = it_ref[...]
    mask = sim > t
    cnt = jnp.sum(mask.astype(jnp.float32), axis=1, keepdims=True)
    ex = jnp.where(mask, jnp.exp((sim - m) * it), 0.0)
    sum_top = jnp.sum(ex, axis=1, keepdims=True) \
        + (KF - cnt) * jnp.exp((t - m) * it)
    total = jnp.exp((ps - m) * it) + sum_top
    losses = ((m - ps) * it + jnp.log(total)) * hw_ref[...]
    part = jnp.sum(losses).reshape(1, 1)

    @pl.when(i == 0)
    def _():
        loss_ref[...] = jnp.zeros((1, 1), jnp.float32)

    loss_ref[...] += part


@jax.jit
def _run(all_emb, w1, b1, w2, b2, dw, hw, pos_rand):
    whole = lambda x: pl.BlockSpec(x.shape, lambda i: (0,) * x.ndim)
    args = (all_emb, w1, b1.reshape(1, 64), w2, b2.reshape(1, 1), dw,
            pos_rand.reshape(N, 1))
    col1 = jax.ShapeDtypeStruct((N, 1), jnp.float32)
    rowspec = pl.BlockSpec((TILE, 1), lambda i: (i, 0))
    sim, m, it, ps, reg = pl.pallas_call(
        _dense_kernel,
        grid=(GRID,),
        in_specs=[whole(a) for a in args],
        out_specs=[
            pl.BlockSpec((TILE, N), lambda i: (i, 0)),
            rowspec, rowspec, rowspec,
            pl.BlockSpec((1, 1), lambda i: (0, 0)),
        ],
        out_shape=[
            jax.ShapeDtypeStruct((N, N), jnp.float32),
            col1, col1, col1,
            jax.ShapeDtypeStruct((1, 1), jnp.float32),
        ],
    )(*args)

    t_edge = _mine(sim)

    fargs = (sim, t_edge.reshape(N, 1), ps, m, it, hw.reshape(N, 1))
    loss_sum = pl.pallas_call(
        _final_kernel,
        grid=(GRID,),
        in_specs=[pl.BlockSpec((TILE, N), lambda i: (i, 0)),
                  rowspec, rowspec, rowspec, rowspec, rowspec],
        out_specs=pl.BlockSpec((1, 1), lambda i: (0, 0)),
        out_shape=jax.ShapeDtypeStruct((1, 1), jnp.float32),
    )(*fargs)
    return loss_sum[0, 0] / N + ALPHA * reg[0, 0]


def kernel(emb_vision, emb_nlp, emb_security, emb_medical, hard_sample_weights,
           W1, b1, W2, b2, domain_weights, domain_ids, pos_rand):
    all_emb = jnp.concatenate([emb_vision, emb_nlp, emb_security, emb_medical],
                              axis=0)
    return _run(all_emb, W1, b1, W2, b2, domain_weights, hard_sample_weights,
                pos_rand)
